# Initial kernel scaffold; baseline (speedup 1.0000x reference)
import jax
import jax.numpy as jnp
from jax.experimental import pallas as pl


def kernel(x, norm1_scale, norm2_scale, Wqkv, bqkv, Wo, bo, Wg, bg, g_rms_w, W1, b1, W2, b2):
    raise NotImplementedError("write your pallas kernel here")


# speedup vs baseline: 2.0380x; 2.0380x over previous
import math

import jax
import jax.numpy as jnp
from jax.experimental import pallas as pl
from jax.experimental.pallas import tpu as pltpu

D = 1024
S = 2048
NHEAD = 16
HD = D // NHEAD
NE = 4
DH = 4096
FCAT = NE * DH
TEMP = 0.5
FBLK = 2048
NFBLK = FCAT // FBLK


def _l2sn(v, scale_row):
    norm = jnp.sqrt(jnp.sum(v * v, axis=-1, keepdims=True))
    return v / (norm * scale_row * (1.0 / math.sqrt(v.shape[-1])) + 1e-8)


def _ln_qkv_kernel(x_ref, scale_ref, wqkv_ref, bqkv_ref, qkv_ref):
    ln1 = _l2sn(x_ref[...], scale_ref[...])
    acc = jnp.dot(ln1.astype(jnp.bfloat16), wqkv_ref[...],
                  preferred_element_type=jnp.float32)
    qkv_ref[...] = (acc + bqkv_ref[...]).astype(jnp.bfloat16)


def _attn_kernel(q_ref, k_ref, v_ref, o_ref):
    q = q_ref[0]
    k = k_ref[0]
    s = jax.lax.dot_general(q, k, (((1,), (1,)), ((), ())),
                            preferred_element_type=jnp.float32)
    s = s * (1.0 / math.sqrt(HD))
    s = s - jnp.max(s, axis=-1, keepdims=True)
    e = jnp.exp(s)
    p = e / jnp.sum(e, axis=-1, keepdims=True)
    o_ref[0] = jnp.dot(p.astype(jnp.bfloat16), v_ref[0],
                       preferred_element_type=jnp.float32).astype(jnp.bfloat16)


def _post_attn_kernel(x_ref, attn_ref, wo_ref, bo_ref, scale2_ref, wg_ref,
                      bg_ref, grms_ref, x1_ref, ln2_ref, gate_ref):
    o = jnp.dot(attn_ref[...], wo_ref[...], preferred_element_type=jnp.float32)
    x1 = x_ref[...] + o + bo_ref[...]
    x1_ref[...] = x1
    ln2 = _l2sn(x1, scale2_ref[...])
    ln2_ref[...] = ln2.astype(jnp.bfloat16)
    logits = jnp.dot(ln2, wg_ref[...], preferred_element_type=jnp.float32)
    logits = logits + bg_ref[...]
    eps = jnp.finfo(jnp.float32).eps
    rms = jnp.sqrt(jnp.mean(logits * logits, axis=-1, keepdims=True) + eps)
    logits = logits / rms * grms_ref[...]
    logits = logits * (1.0 / TEMP)
    m = jnp.max(logits, axis=-1, keepdims=True)
    e = jnp.exp(logits - m)
    g = jnp.max(e, axis=-1, keepdims=True) / jnp.sum(e, axis=-1, keepdims=True)
    gate_ref[...] = g / (g + 1e-6)


def _ffn_kernel(ln2_ref, w1_ref, b1_ref, w2_ref, x1_ref, gate_ref, b2s_ref,
                out_ref):
    j = pl.program_id(0)
    h = jax.lax.dot_general(ln2_ref[...], w1_ref[...], (((1,), (1,)), ((), ())),
                            preferred_element_type=jnp.float32)
    h = h + b1_ref[0]
    h = 0.5 * h * (1.0 + jax.lax.erf(h * (1.0 / math.sqrt(2.0))))
    part = jnp.dot(h.astype(jnp.bfloat16), w2_ref[...],
                   preferred_element_type=jnp.float32)

    @pl.when(j == 0)
    def _():
        out_ref[...] = part

    @pl.when(j > 0)
    def _():
        out_ref[...] += part

    @pl.when(j == NFBLK - 1)
    def _():
        m = out_ref[...] + b2s_ref[...]
        out_ref[...] = x1_ref[...] + gate_ref[...] * m


def kernel(x, norm1_scale, norm2_scale, Wqkv, bqkv, Wo, bo, Wg, bg, g_rms_w,
           W1, b1, W2, b2):
    b, s, d = x.shape
    x2 = x.reshape(s, d)
    scale1 = norm1_scale.reshape(1, d)
    scale2 = norm2_scale.reshape(1, d)
    wqkv_t = Wqkv.T.astype(jnp.bfloat16)
    bqkv_r = bqkv.reshape(1, 3 * D)
    wo_t = Wo.T.astype(jnp.bfloat16)
    bo_r = bo.reshape(1, D)
    wg_t = Wg.T
    bg_r = bg.reshape(1, NE)
    grms_r = g_rms_w.reshape(1, NE)
    w1c = W1.reshape(FCAT, d).astype(jnp.bfloat16)
    b1c = b1.reshape(NFBLK, 1, FBLK)
    w2c = W2.transpose(0, 2, 1).reshape(FCAT, d).astype(jnp.bfloat16)
    b2s = b2.sum(axis=0).reshape(1, D)

    qkv = pl.pallas_call(
        _ln_qkv_kernel,
        out_shape=jax.ShapeDtypeStruct((S, 3 * D), jnp.bfloat16),
    )(x2, scale1, wqkv_t, bqkv_r)

    q = qkv[:, :D].reshape(S, NHEAD, HD).transpose(1, 0, 2)
    k = qkv[:, D:2 * D].reshape(S, NHEAD, HD).transpose(1, 0, 2)
    v = qkv[:, 2 * D:].reshape(S, NHEAD, HD).transpose(1, 0, 2)

    attn = pl.pallas_call(
        _attn_kernel,
        grid=(NHEAD,),
        in_specs=[
            pl.BlockSpec((1, S, HD), lambda h: (h, 0, 0)),
            pl.BlockSpec((1, S, HD), lambda h: (h, 0, 0)),
            pl.BlockSpec((1, S, HD), lambda h: (h, 0, 0)),
        ],
        out_specs=pl.BlockSpec((1, S, HD), lambda h: (h, 0, 0)),
        out_shape=jax.ShapeDtypeStruct((NHEAD, S, HD), jnp.bfloat16),
    )(q, k, v)
    attn = attn.transpose(1, 0, 2).reshape(S, D)

    x1, ln2, gate = pl.pallas_call(
        _post_attn_kernel,
        out_shape=(
            jax.ShapeDtypeStruct((S, D), jnp.float32),
            jax.ShapeDtypeStruct((S, D), jnp.bfloat16),
            jax.ShapeDtypeStruct((S, 1), jnp.float32),
        ),
    )(x2, attn, wo_t, bo_r, scale2, wg_t, bg_r, grms_r)

    out = pl.pallas_call(
        _ffn_kernel,
        grid=(NFBLK,),
        in_specs=[
            pl.BlockSpec((S, D), lambda j: (0, 0)),
            pl.BlockSpec((FBLK, D), lambda j: (j, 0)),
            pl.BlockSpec((1, 1, FBLK), lambda j: (j, 0, 0)),
            pl.BlockSpec((FBLK, D), lambda j: (j, 0)),
            pl.BlockSpec((S, D), lambda j: (0, 0)),
            pl.BlockSpec((S, 1), lambda j: (0, 0)),
            pl.BlockSpec((1, D), lambda j: (0, 0)),
        ],
        out_specs=pl.BlockSpec((S, D), lambda j: (0, 0)),
        out_shape=jax.ShapeDtypeStruct((S, D), jnp.float32),
    )(ln2, w1c, b1c, w2c, x1, gate, b2s)

    return out.reshape(b, s, d)


# speedup vs baseline: 2.9174x; 1.4315x over previous
import math

import jax
import jax.numpy as jnp
from jax.experimental import pallas as pl
from jax.experimental.pallas import tpu as pltpu

D = 1024
S = 2048
NHEAD = 16
HD = D // NHEAD
NE = 4
DH = 4096
FCAT = NE * DH
TEMP = 0.5
FBLK = 1024
NFBLK = FCAT // FBLK


def _l2sn(v, scale_row):
    norm = jnp.sqrt(jnp.sum(v * v, axis=-1, keepdims=True))
    return v / (norm * scale_row * (1.0 / math.sqrt(v.shape[-1])) + 1e-8)


def _ln_qkv_kernel(x_ref, scale_ref, wqkv_ref, bqkv_ref, qkv_ref):
    ln1 = _l2sn(x_ref[...], scale_ref[...])
    w = wqkv_ref[...].astype(jnp.bfloat16)
    acc = jax.lax.dot_general(
        ln1.astype(jnp.bfloat16), w,
        (((1,), (1,)), ((), ())), preferred_element_type=jnp.float32)
    qkv_ref[...] = (acc + bqkv_ref[...]).astype(jnp.bfloat16)


def _attn_one_head(q, k, v):
    s = jax.lax.dot_general(q, k, (((1,), (1,)), ((), ())),
                            preferred_element_type=jnp.float32)
    s = s * (1.0 / math.sqrt(HD))
    s = s - jnp.max(s, axis=-1, keepdims=True)
    e = jnp.exp(s)
    p = e * (1.0 / jnp.sum(e, axis=-1, keepdims=True))
    return jnp.dot(p.astype(jnp.bfloat16), v,
                   preferred_element_type=jnp.float32)


def _attn_kernel(q_ref, k_ref, v_ref, o_ref):
    q = q_ref[...]
    k = k_ref[...]
    v = v_ref[...]
    o0 = _attn_one_head(q[:, :HD], k[:, :HD], v[:, :HD])
    o1 = _attn_one_head(q[:, HD:], k[:, HD:], v[:, HD:])
    o_ref[...] = jnp.concatenate([o0, o1], axis=1).astype(jnp.bfloat16)


def _post_attn_kernel(x_ref, attn_ref, wo_ref, bo_ref, scale2_ref, wg_ref,
                      bg_ref, grms_ref, x1_ref, ln2_ref, gate_ref):
    wo = wo_ref[...].astype(jnp.bfloat16)
    o = jax.lax.dot_general(attn_ref[...], wo, (((1,), (1,)), ((), ())),
                            preferred_element_type=jnp.float32)
    x1 = x_ref[...] + o + bo_ref[...]
    x1_ref[...] = x1
    ln2 = _l2sn(x1, scale2_ref[...])
    ln2b = ln2.astype(jnp.bfloat16)
    ln2_ref[...] = ln2b
    logits = jax.lax.dot_general(ln2, wg_ref[...], (((1,), (1,)), ((), ())),
                                 preferred_element_type=jnp.float32)
    logits = logits + bg_ref[...]
    eps = jnp.finfo(jnp.float32).eps
    rms = jnp.sqrt(jnp.mean(logits * logits, axis=-1, keepdims=True) + eps)
    logits = logits / rms * grms_ref[...]
    logits = logits * (1.0 / TEMP)
    m = jnp.max(logits, axis=-1, keepdims=True)
    e = jnp.exp(logits - m)
    g = jnp.max(e, axis=-1, keepdims=True) / jnp.sum(e, axis=-1, keepdims=True)
    gate_ref[...] = g / (g + 1e-6)


def _ffn_kernel(ln2_ref, w1_ref, b1_ref, w2_ref, x1_ref, gate_ref, b2s_ref,
                out_ref):
    j = pl.program_id(0)
    w1 = w1_ref[...].astype(jnp.bfloat16)
    h = jax.lax.dot_general(ln2_ref[...], w1, (((1,), (1,)), ((), ())),
                            preferred_element_type=jnp.float32)
    h = h + b1_ref[0]
    h = 0.5 * h * (1.0 + jax.lax.erf(h * (1.0 / math.sqrt(2.0))))
    w2 = w2_ref[0].astype(jnp.bfloat16)
    part = jax.lax.dot_general(h.astype(jnp.bfloat16), w2,
                               (((1,), (1,)), ((), ())),
                               preferred_element_type=jnp.float32)

    @pl.when(j == 0)
    def _():
        out_ref[...] = part

    @pl.when(j > 0)
    def _():
        out_ref[...] += part

    @pl.when(j == NFBLK - 1)
    def _():
        m = out_ref[...] + b2s_ref[...]
        out_ref[...] = x1_ref[...] + gate_ref[...] * m


def kernel(x, norm1_scale, norm2_scale, Wqkv, bqkv, Wo, bo, Wg, bg, g_rms_w,
           W1, b1, W2, b2):
    b, s, d = x.shape
    x2 = x.reshape(s, d)
    scale1 = norm1_scale.reshape(1, d)
    scale2 = norm2_scale.reshape(1, d)
    bqkv_r = bqkv.reshape(1, 3 * D)
    bo_r = bo.reshape(1, D)
    wg_t = Wg
    bg_r = bg.reshape(1, NE)
    grms_r = g_rms_w.reshape(1, NE)
    w1c = W1.reshape(FCAT, d)
    b1c = b1.reshape(NFBLK, 1, FBLK)
    w2c = W2.reshape(NE, D, DH)
    b2s = b2.sum(axis=0).reshape(1, D)
    fperq = DH // FBLK

    qkv = pl.pallas_call(
        _ln_qkv_kernel,
        out_shape=jax.ShapeDtypeStruct((S, 3 * D), jnp.bfloat16),
    )(x2, scale1, Wqkv, bqkv_r)

    attn = pl.pallas_call(
        _attn_kernel,
        grid=(NHEAD // 2,),
        in_specs=[
            pl.BlockSpec((S, 2 * HD), lambda h: (0, h)),
            pl.BlockSpec((S, 2 * HD), lambda h: (0, 8 + h)),
            pl.BlockSpec((S, 2 * HD), lambda h: (0, 16 + h)),
        ],
        out_specs=pl.BlockSpec((S, 2 * HD), lambda h: (0, h)),
        out_shape=jax.ShapeDtypeStruct((S, D), jnp.bfloat16),
    )(qkv, qkv, qkv)

    x1, ln2, gate = pl.pallas_call(
        _post_attn_kernel,
        out_shape=(
            jax.ShapeDtypeStruct((S, D), jnp.float32),
            jax.ShapeDtypeStruct((S, D), jnp.bfloat16),
            jax.ShapeDtypeStruct((S, 1), jnp.float32),
        ),
    )(x2, attn, Wo, bo_r, scale2, wg_t, bg_r, grms_r)

    out = pl.pallas_call(
        _ffn_kernel,
        grid=(NFBLK,),
        in_specs=[
            pl.BlockSpec((S, D), lambda j: (0, 0)),
            pl.BlockSpec((FBLK, D), lambda j: (j, 0)),
            pl.BlockSpec((1, 1, FBLK), lambda j: (j, 0, 0)),
            pl.BlockSpec((1, D, FBLK), lambda j: (j // fperq, 0, j % fperq)),
            pl.BlockSpec((S, D), lambda j: (0, 0)),
            pl.BlockSpec((S, 1), lambda j: (0, 0)),
            pl.BlockSpec((1, D), lambda j: (0, 0)),
        ],
        out_specs=pl.BlockSpec((S, D), lambda j: (0, 0)),
        out_shape=jax.ShapeDtypeStruct((S, D), jnp.float32),
    )(ln2, w1c, b1c, w2c, x1, gate, b2s)

    return out.reshape(b, s, d)


# speedup vs baseline: 3.7561x; 1.2875x over previous
import math

import jax
import jax.numpy as jnp
from jax.experimental import pallas as pl
from jax.experimental.pallas import tpu as pltpu

D = 1024
S = 2048
NHEAD = 16
HD = D // NHEAD
NE = 4
DH = 4096
FCAT = NE * DH
TEMP = 0.5
FBLK = 1024
NFBLK = FCAT // FBLK
QSCALE = math.log2(math.e) / math.sqrt(HD)


def _l2sn(v, scale_row):
    norm = jnp.sqrt(jnp.sum(v * v, axis=-1, keepdims=True))
    return v / (norm * scale_row * (1.0 / math.sqrt(v.shape[-1])) + 1e-8)


def _attn_one_head(q, k, v_aug, m_row):
    sT = jax.lax.dot_general(k, q, (((1,), (1,)), ((), ())),
                             preferred_element_type=jnp.float32)
    e = jnp.exp2(jnp.maximum(sT - m_row, -120.0)).astype(jnp.bfloat16)
    oT = jax.lax.dot_general(v_aug, e, (((0,), (0,)), ((), ())),
                             preferred_element_type=jnp.float32)
    denom = oT[HD:HD + 1]
    return (oT[:HD] * (1.0 / denom)).astype(jnp.bfloat16)


def _attn_kernel(x_ref, scale_ref, wq_ref, wk_ref, wv_ref, bq_ref, bk_ref,
                 bv_ref, o_ref, ln1_ref):
    h = pl.program_id(0)

    @pl.when(h == 0)
    def _():
        ln1_ref[...] = _l2sn(x_ref[...], scale_ref[...]).astype(jnp.bfloat16)

    ln1 = ln1_ref[...]
    wq = wq_ref[...].astype(jnp.bfloat16)
    wk = wk_ref[...].astype(jnp.bfloat16)
    wv = wv_ref[...].astype(jnp.bfloat16)
    qf = jax.lax.dot_general(ln1, wq, (((1,), (1,)), ((), ())),
                             preferred_element_type=jnp.float32)
    qf = (qf + bq_ref[0]) * QSCALE
    q = qf.astype(jnp.bfloat16)
    kf = jax.lax.dot_general(ln1, wk, (((1,), (1,)), ((), ())),
                             preferred_element_type=jnp.float32)
    kf = kf + bk_ref[0]
    k = kf.astype(jnp.bfloat16)
    v = jax.lax.dot_general(ln1, wv, (((1,), (1,)), ((), ())),
                            preferred_element_type=jnp.float32)
    v = (v + bv_ref[0]).astype(jnp.bfloat16)
    ones_col = jnp.ones((S, 1), jnp.bfloat16)
    q2 = qf * qf
    k2 = kf * kf
    outs = []
    for i in range(2):
        sl = slice(i * HD, (i + 1) * HD)
        qn2 = jnp.sum(q2[:, sl], axis=1, keepdims=True)
        kn2max = jnp.max(jnp.sum(k2[:, sl], axis=1))
        m_row = jnp.sqrt(qn2 * kn2max).T + 1e-3
        v_aug = jnp.concatenate([v[:, sl], ones_col], axis=1)
        outs.append(_attn_one_head(q[:, sl], k[:, sl], v_aug, m_row))
    o_ref[...] = jnp.concatenate(outs, axis=0)


def _post_attn_kernel(x_ref, attn_ref, wo_ref, bo_ref, scale2_ref, wg_ref,
                      bg_ref, grms_ref, x1_ref, ln2_ref, gate_ref):
    wo = wo_ref[...].astype(jnp.bfloat16)
    o = jax.lax.dot_general(attn_ref[...], wo, (((0,), (1,)), ((), ())),
                            preferred_element_type=jnp.float32)
    x1 = x_ref[...] + o + bo_ref[...]
    x1_ref[...] = x1
    ln2 = _l2sn(x1, scale2_ref[...])
    ln2b = ln2.astype(jnp.bfloat16)
    ln2_ref[...] = ln2b
    logits = jax.lax.dot_general(ln2, wg_ref[...], (((1,), (1,)), ((), ())),
                                 preferred_element_type=jnp.float32)
    logits = logits + bg_ref[...]
    eps = jnp.finfo(jnp.float32).eps
    rms = jnp.sqrt(jnp.mean(logits * logits, axis=-1, keepdims=True) + eps)
    logits = logits / rms * grms_ref[...]
    logits = logits * (1.0 / TEMP)
    m = jnp.max(logits, axis=-1, keepdims=True)
    e = jnp.exp(logits - m)
    g = jnp.max(e, axis=-1, keepdims=True) / jnp.sum(e, axis=-1, keepdims=True)
    gate_ref[...] = g / (g + 1e-6)


def _ffn_kernel(ln2_ref, w1_ref, b1_ref, w2_ref, x1_ref, gate_ref, b2s_ref,
                out_ref):
    j = pl.program_id(0)
    w1 = w1_ref[...].astype(jnp.bfloat16)
    h = jax.lax.dot_general(ln2_ref[...], w1, (((1,), (1,)), ((), ())),
                            preferred_element_type=jnp.float32)
    h = h + b1_ref[0]
    h = 0.5 * h * (1.0 + jax.lax.erf(h * (1.0 / math.sqrt(2.0))))
    w2 = w2_ref[0].astype(jnp.bfloat16)
    part = jax.lax.dot_general(h.astype(jnp.bfloat16), w2,
                               (((1,), (1,)), ((), ())),
                               preferred_element_type=jnp.float32)

    @pl.when(j == 0)
    def _():
        out_ref[...] = part

    @pl.when(j > 0)
    def _():
        out_ref[...] += part

    @pl.when(j == NFBLK - 1)
    def _():
        m = out_ref[...] + b2s_ref[...]
        out_ref[...] = x1_ref[...] + gate_ref[...] * m


def kernel(x, norm1_scale, norm2_scale, Wqkv, bqkv, Wo, bo, Wg, bg, g_rms_w,
           W1, b1, W2, b2):
    b, s, d = x.shape
    x2 = x.reshape(s, d)
    scale1 = norm1_scale.reshape(1, d)
    scale2 = norm2_scale.reshape(1, d)
    bqkv_r = bqkv.reshape(3 * NHEAD // 2, 1, 2 * HD)
    bo_r = bo.reshape(1, D)
    bg_r = bg.reshape(1, NE)
    grms_r = g_rms_w.reshape(1, NE)
    w1c = W1.reshape(FCAT, d)
    b1c = b1.reshape(NFBLK, 1, FBLK)
    w2c = W2.reshape(NE, D, DH)
    b2s = b2.sum(axis=0).reshape(1, D)
    fperq = DH // FBLK
    npair = NHEAD // 2

    attn = pl.pallas_call(
        _attn_kernel,
        grid=(npair,),
        in_specs=[
            pl.BlockSpec((S, D), lambda h: (0, 0)),
            pl.BlockSpec((1, D), lambda h: (0, 0)),
            pl.BlockSpec((2 * HD, D), lambda h: (h, 0)),
            pl.BlockSpec((2 * HD, D), lambda h: (npair + h, 0)),
            pl.BlockSpec((2 * HD, D), lambda h: (2 * npair + h, 0)),
            pl.BlockSpec((1, 1, 2 * HD), lambda h: (h, 0, 0)),
            pl.BlockSpec((1, 1, 2 * HD), lambda h: (npair + h, 0, 0)),
            pl.BlockSpec((1, 1, 2 * HD), lambda h: (2 * npair + h, 0, 0)),
        ],
        out_specs=pl.BlockSpec((2 * HD, S), lambda h: (h, 0)),
        out_shape=jax.ShapeDtypeStruct((D, S), jnp.bfloat16),
        scratch_shapes=[pltpu.VMEM((S, D), jnp.bfloat16)],
    )(x2, scale1, Wqkv, Wqkv, Wqkv, bqkv_r, bqkv_r, bqkv_r)

    x1, ln2, gate = pl.pallas_call(
        _post_attn_kernel,
        out_shape=(
            jax.ShapeDtypeStruct((S, D), jnp.float32),
            jax.ShapeDtypeStruct((S, D), jnp.bfloat16),
            jax.ShapeDtypeStruct((S, 1), jnp.float32),
        ),
    )(x2, attn, Wo, bo_r, scale2, Wg, bg_r, grms_r)

    out = pl.pallas_call(
        _ffn_kernel,
        grid=(NFBLK,),
        in_specs=[
            pl.BlockSpec((S, D), lambda j: (0, 0)),
            pl.BlockSpec((FBLK, D), lambda j: (j, 0)),
            pl.BlockSpec((1, 1, FBLK), lambda j: (j, 0, 0)),
            pl.BlockSpec((1, D, FBLK), lambda j: (j // fperq, 0, j % fperq)),
            pl.BlockSpec((S, D), lambda j: (0, 0)),
            pl.BlockSpec((S, 1), lambda j: (0, 0)),
            pl.BlockSpec((1, D), lambda j: (0, 0)),
        ],
        out_specs=pl.BlockSpec((S, D), lambda j: (0, 0)),
        out_shape=jax.ShapeDtypeStruct((S, D), jnp.float32),
    )(ln2, w1c, b1c, w2c, x1, gate, b2s)

    return out.reshape(b, s, d)


# speedup vs baseline: 3.7675x; 1.0031x over previous
import math

import jax
import jax.numpy as jnp
from jax.experimental import pallas as pl
from jax.experimental.pallas import tpu as pltpu

D = 1024
S = 2048
NHEAD = 16
HD = D // NHEAD
NE = 4
DH = 4096
FCAT = NE * DH
TEMP = 0.5
FBLK = 1024
NFBLK = FCAT // FBLK
QSCALE = math.log2(math.e) / math.sqrt(HD)


def _l2sn(v, scale_row):
    norm = jnp.sqrt(jnp.sum(v * v, axis=-1, keepdims=True))
    return v / (norm * scale_row * (1.0 / math.sqrt(v.shape[-1])) + 1e-8)


def _attn_one_head(q, k, v_aug, m_row):
    sT = jax.lax.dot_general(k, q, (((1,), (1,)), ((), ())),
                             preferred_element_type=jnp.float32)
    e = jnp.exp2(jnp.maximum(sT - m_row, -120.0)).astype(jnp.bfloat16)
    oT = jax.lax.dot_general(v_aug, e, (((0,), (0,)), ((), ())),
                             preferred_element_type=jnp.float32)
    denom = oT[HD:HD + 1]
    return (oT[:HD] * (1.0 / denom)).astype(jnp.bfloat16)


def _attn_post_kernel(x_ref, scale_ref, wq_ref, wk_ref, wv_ref, bq_ref,
                      bk_ref, bv_ref, wo_ref, bo_ref, scale2_ref, wg_ref,
                      bg_ref, grms_ref, x1_ref, ln2_ref, gate_ref,
                      ln1_ref, at_ref):
    h = pl.program_id(0)

    @pl.when(h == 0)
    def _():
        ln1_ref[...] = _l2sn(x_ref[...], scale_ref[...]).astype(jnp.bfloat16)

    @pl.when(h < NHEAD // 2)
    def _():
        ln1 = ln1_ref[...]
        wq = wq_ref[...].astype(jnp.bfloat16)
        wk = wk_ref[...].astype(jnp.bfloat16)
        wv = wv_ref[...].astype(jnp.bfloat16)
        qf = jax.lax.dot_general(ln1, wq, (((1,), (1,)), ((), ())),
                                 preferred_element_type=jnp.float32)
        qf = (qf + bq_ref[0]) * QSCALE
        q = qf.astype(jnp.bfloat16)
        kf = jax.lax.dot_general(ln1, wk, (((1,), (1,)), ((), ())),
                                 preferred_element_type=jnp.float32)
        kf = kf + bk_ref[0]
        k = kf.astype(jnp.bfloat16)
        v = jax.lax.dot_general(ln1, wv, (((1,), (1,)), ((), ())),
                                preferred_element_type=jnp.float32)
        v = (v + bv_ref[0]).astype(jnp.bfloat16)
        ones_col = jnp.ones((S, 1), jnp.bfloat16)
        q2 = qf * qf
        k2 = kf * kf
        for i in range(2):
            sl = slice(i * HD, (i + 1) * HD)
            qn2 = jnp.sum(q2[:, sl], axis=1, keepdims=True)
            kn2max = jnp.max(jnp.sum(k2[:, sl], axis=1))
            m_col = jnp.sqrt(qn2 * kn2max) + 1e-3
            v_aug = jnp.concatenate([v[:, sl], ones_col], axis=1)
            for half in range(2):
                qh = slice(half * (S // 2), (half + 1) * (S // 2))
                o_part = _attn_one_head(q[qh, sl], k[:, sl], v_aug,
                                        m_col[qh].T)
                at_ref[pl.ds(h * 2 * HD + i * HD, HD),
                       pl.ds(half * (S // 2), S // 2)] = o_part

    @pl.when(h == NHEAD // 2)
    def _():
        wo = wo_ref[...].astype(jnp.bfloat16)
        o = jax.lax.dot_general(at_ref[...], wo, (((0,), (1,)), ((), ())),
                                preferred_element_type=jnp.float32)
        x1 = x_ref[...] + o + bo_ref[...]
        x1_ref[...] = x1
        ln2 = _l2sn(x1, scale2_ref[...])
        ln2_ref[...] = ln2.astype(jnp.bfloat16)
        logits = jax.lax.dot_general(ln2, wg_ref[...], (((1,), (1,)), ((), ())),
                                     preferred_element_type=jnp.float32)
        logits = logits + bg_ref[...]
        eps = jnp.finfo(jnp.float32).eps
        rms = jnp.sqrt(jnp.mean(logits * logits, axis=-1, keepdims=True) + eps)
        logits = logits / rms * grms_ref[...]
        logits = logits * (1.0 / TEMP)
        m = jnp.max(logits, axis=-1, keepdims=True)
        e = jnp.exp(logits - m)
        g = (jnp.max(e, axis=-1, keepdims=True)
             / jnp.sum(e, axis=-1, keepdims=True))
        gate_ref[...] = g / (g + 1e-6)


def _ffn_kernel(ln2_ref, w1_ref, b1_ref, w2_ref, x1_ref, gate_ref, b2s_ref,
                out_ref):
    j = pl.program_id(0)
    w1 = w1_ref[...].astype(jnp.bfloat16)
    h = jax.lax.dot_general(ln2_ref[...], w1, (((1,), (1,)), ((), ())),
                            preferred_element_type=jnp.float32)
    hf = h + b1_ref[0]
    hb = (0.5 * hf * (1.0 + jax.lax.erf(hf * (1.0 / math.sqrt(2.0))))
          ).astype(jnp.bfloat16)
    w2 = w2_ref[0].astype(jnp.bfloat16)
    part = jax.lax.dot_general(hb, w2,
                               (((1,), (1,)), ((), ())),
                               preferred_element_type=jnp.float32)

    @pl.when(j == 0)
    def _():
        out_ref[...] = part

    @pl.when(j > 0)
    def _():
        out_ref[...] += part

    @pl.when(j == NFBLK - 1)
    def _():
        m = out_ref[...] + b2s_ref[...]
        out_ref[...] = x1_ref[...] + gate_ref[...] * m


def kernel(x, norm1_scale, norm2_scale, Wqkv, bqkv, Wo, bo, Wg, bg, g_rms_w,
           W1, b1, W2, b2):
    b, s, d = x.shape
    x2 = x.reshape(s, d)
    scale1 = norm1_scale.reshape(1, d)
    scale2 = norm2_scale.reshape(1, d)
    bqkv_r = bqkv.reshape(3 * NHEAD // 2, 1, 2 * HD)
    bo_r = bo.reshape(1, D)
    bg_r = bg.reshape(1, NE)
    grms_r = g_rms_w.reshape(1, NE)
    w1c = W1.reshape(FCAT, d)
    b1c = b1.reshape(NFBLK, 1, FBLK)
    w2c = W2.reshape(NE, D, DH)
    b2s = b2.sum(axis=0).reshape(1, D)
    fperq = DH // FBLK
    npair = NHEAD // 2

    hcap = npair - 1

    x1, ln2, gate = pl.pallas_call(
        _attn_post_kernel,
        grid=(npair + 1,),
        in_specs=[
            pl.BlockSpec((S, D), lambda h: (0, 0)),
            pl.BlockSpec((1, D), lambda h: (0, 0)),
            pl.BlockSpec((2 * HD, D), lambda h: (jnp.minimum(h, hcap), 0)),
            pl.BlockSpec((2 * HD, D),
                         lambda h: (npair + jnp.minimum(h, hcap), 0)),
            pl.BlockSpec((2 * HD, D),
                         lambda h: (2 * npair + jnp.minimum(h, hcap), 0)),
            pl.BlockSpec((1, 1, 2 * HD),
                         lambda h: (jnp.minimum(h, hcap), 0, 0)),
            pl.BlockSpec((1, 1, 2 * HD),
                         lambda h: (npair + jnp.minimum(h, hcap), 0, 0)),
            pl.BlockSpec((1, 1, 2 * HD),
                         lambda h: (2 * npair + jnp.minimum(h, hcap), 0, 0)),
            pl.BlockSpec((D, D), lambda h: (0, 0)),
            pl.BlockSpec((1, D), lambda h: (0, 0)),
            pl.BlockSpec((1, D), lambda h: (0, 0)),
            pl.BlockSpec((NE, D), lambda h: (0, 0)),
            pl.BlockSpec((1, NE), lambda h: (0, 0)),
            pl.BlockSpec((1, NE), lambda h: (0, 0)),
        ],
        out_specs=(
            pl.BlockSpec((S, D), lambda h: (0, 0)),
            pl.BlockSpec((S, D), lambda h: (0, 0)),
            pl.BlockSpec((S, 1), lambda h: (0, 0)),
        ),
        out_shape=(
            jax.ShapeDtypeStruct((S, D), jnp.float32),
            jax.ShapeDtypeStruct((S, D), jnp.bfloat16),
            jax.ShapeDtypeStruct((S, 1), jnp.float32),
        ),
        scratch_shapes=[pltpu.VMEM((S, D), jnp.bfloat16),
                        pltpu.VMEM((D, S), jnp.bfloat16)],
    )(x2, scale1, Wqkv, Wqkv, Wqkv, bqkv_r, bqkv_r, bqkv_r,
      Wo, bo_r, scale2, Wg, bg_r, grms_r)

    out = pl.pallas_call(
        _ffn_kernel,
        grid=(NFBLK,),
        in_specs=[
            pl.BlockSpec((S, D), lambda j: (0, 0)),
            pl.BlockSpec((FBLK, D), lambda j: (j, 0)),
            pl.BlockSpec((1, 1, FBLK), lambda j: (j, 0, 0)),
            pl.BlockSpec((1, D, FBLK), lambda j: (j // fperq, 0, j % fperq)),
            pl.BlockSpec((S, D), lambda j: (0, 0)),
            pl.BlockSpec((S, 1), lambda j: (0, 0)),
            pl.BlockSpec((1, D), lambda j: (0, 0)),
        ],
        out_specs=pl.BlockSpec((S, D), lambda j: (0, 0)),
        out_shape=jax.ShapeDtypeStruct((S, D), jnp.float32),
    )(ln2, w1c, b1c, w2c, x1, gate, b2s)

    return out.reshape(b, s, d)
